# initial kernel scaffold (unmeasured)
import jax
import jax.numpy as jnp
from jax import lax
from jax.experimental import pallas as pl
from jax.experimental.pallas import tpu as pltpu

N_DEV = 8
SQ = 1024
D = 1024
HQ = 8
DH = 128
SCALE = 0.08838834764831843


def _body(x_ref, wq_ref, k_ref, v_ref, wo_ref, out_ref,
          comm_ref, send_sems, recv_sems):
    my_pos = lax.axis_index("i")
    left = lax.rem(my_pos - 1 + N_DEV, N_DEV)
    right = lax.rem(my_pos + 1, N_DEV)

    barrier_sem = pltpu.get_barrier_semaphore()
    for nbr in (left, right):
        pl.semaphore_signal(
            barrier_sem, inc=1,
            device_id=(nbr,), device_id_type=pl.DeviceIdType.MESH,
        )
    pl.semaphore_wait(barrier_sem, 2)

    q = jnp.dot(x_ref[...], wq_ref[...],
                preferred_element_type=jnp.float32)

    rows = lax.broadcasted_iota(jnp.int32, (SQ, SQ), 0)
    cols = lax.broadcasted_iota(jnp.int32, (SQ, SQ), 1)
    qb = rows // 64
    kb = cols // 64
    mask = (qb == kb) | (kb == 0) | (lax.rem(qb + kb, 3) == 0)

    ctx = jnp.zeros((SQ, D), dtype=jnp.float32)
    for h in range(HQ):
        q_h = q[:, h * DH:(h + 1) * DH]
        k_h = k_ref[:, h, :]
        v_h = v_ref[:, h, :]
        scores = lax.dot_general(
            q_h, k_h, (((1,), (1,)), ((), ())),
            preferred_element_type=jnp.float32) * SCALE
        scores = jnp.where(mask, scores, -1e9)
        m = jnp.max(scores, axis=1, keepdims=True)
        e = jnp.exp(scores - m)
        s = jnp.sum(e, axis=1, keepdims=True)
        w = e / s
        ctx_h = jnp.dot(w, v_h, preferred_element_type=jnp.float32)
        ctx = ctx.at[:, h * DH:(h + 1) * DH].set(ctx_h)

    comm_ref[0] = jnp.dot(ctx, wo_ref[...],
                          preferred_element_type=jnp.float32)

    for h in range(N_DEV - 1):
        rdma = pltpu.make_async_remote_copy(
            src_ref=comm_ref.at[h],
            dst_ref=comm_ref.at[h + 1],
            send_sem=send_sems.at[h],
            recv_sem=recv_sems.at[h],
            device_id=(right,),
            device_id_type=pl.DeviceIdType.MESH,
        )
        rdma.start()
        rdma.wait()

    acc = comm_ref[0]
    for s in range(1, N_DEV):
        acc = acc + comm_ref[s]
    out_ref[...] = acc


def kernel(x, Wq, K_ext, V_ext, Wo):
    pos = lax.axis_index("i")
    x2 = x[0]
    k2 = K_ext[0]
    v2 = V_ext[0]
    wq_s = lax.dynamic_slice(Wq, (0, pos * D), (1024, D))
    wo_s = lax.dynamic_slice(Wo, (pos * D, 0), (D, 1024))

    out = pl.pallas_call(
        _body,
        out_shape=jax.ShapeDtypeStruct((SQ, 1024), jnp.float32),
        in_specs=[pl.BlockSpec(memory_space=pltpu.VMEM)] * 5,
        out_specs=pl.BlockSpec(memory_space=pltpu.VMEM),
        scratch_shapes=[
            pltpu.VMEM((N_DEV, SQ, 1024), jnp.float32),
            pltpu.SemaphoreType.DMA((N_DEV - 1,)),
            pltpu.SemaphoreType.DMA((N_DEV - 1,)),
        ],
        compiler_params=pltpu.CompilerParams(
            collective_id=0,
            vmem_limit_bytes=128 * 1024 * 1024,
        ),
    )(x2, wq_s, k2, v2, wo_s)
    return out[None]


# baseline (device time: 152654 ns/iter reference)
import jax
import jax.numpy as jnp
from jax import lax
from jax.experimental import pallas as pl
from jax.experimental.pallas import tpu as pltpu

N_DEV = 8
SQ = 1024
D = 1024
HQ = 8
DH = 128
SCALE = 0.08838834764831843


CHUNK = SQ // N_DEV


def _body(x_ref, wq_ref, k_ref, v_ref, wo_ref, out_ref,
          rbuf_ref, rs_send, rs_recv, ag_send, ag_recv):
    my_pos = lax.axis_index("i")
    left = lax.rem(my_pos - 1 + N_DEV, N_DEV)
    right = lax.rem(my_pos + 1, N_DEV)

    barrier_sem = pltpu.get_barrier_semaphore()
    for nbr in (left, right):
        pl.semaphore_signal(
            barrier_sem, inc=1,
            device_id=(nbr,), device_id_type=pl.DeviceIdType.MESH,
        )
    pl.semaphore_wait(barrier_sem, 2)

    q = jnp.dot(x_ref[...], wq_ref[...],
                preferred_element_type=jnp.float32)

    rows = lax.broadcasted_iota(jnp.int32, (SQ, SQ), 0)
    cols = lax.broadcasted_iota(jnp.int32, (SQ, SQ), 1)
    qb = rows // 64
    kb = cols // 64
    mask = (qb == kb) | (kb == 0) | (lax.rem(qb + kb, 3) == 0)

    partial = jnp.zeros((SQ, 1024), dtype=jnp.float32)
    for h in range(HQ):
        q_h = q[:, h * DH:(h + 1) * DH]
        k_h = k_ref[:, h, :]
        v_h = v_ref[:, h, :]
        scores = lax.dot_general(
            q_h, k_h, (((1,), (1,)), ((), ())),
            preferred_element_type=jnp.float32) * SCALE
        scores = jnp.where(mask, scores, -1e9)
        m = jnp.max(scores, axis=1, keepdims=True)
        e = jnp.exp(scores - m)
        s = jnp.sum(e, axis=1, keepdims=True)
        w = e / s
        ctx_h = jnp.dot(w, v_h, preferred_element_type=jnp.float32)
        partial = partial + jnp.dot(
            ctx_h, wo_ref[h * DH:(h + 1) * DH, :],
            preferred_element_type=jnp.float32)

    out_ref[...] = partial

    for s in range(N_DEV - 1):
        c_send = lax.rem(my_pos - s + N_DEV, N_DEV)
        rdma = pltpu.make_async_remote_copy(
            src_ref=out_ref.at[pl.ds(c_send * CHUNK, CHUNK), :],
            dst_ref=rbuf_ref.at[s],
            send_sem=rs_send.at[s],
            recv_sem=rs_recv.at[s],
            device_id=(right,),
            device_id_type=pl.DeviceIdType.MESH,
        )
        rdma.start()
        rdma.wait()
        c_recv = lax.rem(my_pos - s - 1 + N_DEV, N_DEV)
        out_ref[pl.ds(c_recv * CHUNK, CHUNK), :] = (
            out_ref[pl.ds(c_recv * CHUNK, CHUNK), :] + rbuf_ref[s]
        )

    for s in range(N_DEV - 1):
        g_send = lax.rem(my_pos + 1 - s + 2 * N_DEV, N_DEV)
        rdma = pltpu.make_async_remote_copy(
            src_ref=out_ref.at[pl.ds(g_send * CHUNK, CHUNK), :],
            dst_ref=out_ref.at[pl.ds(g_send * CHUNK, CHUNK), :],
            send_sem=ag_send.at[s],
            recv_sem=ag_recv.at[s],
            device_id=(right,),
            device_id_type=pl.DeviceIdType.MESH,
        )
        rdma.start()
        rdma.wait()


def kernel(x, Wq, K_ext, V_ext, Wo):
    pos = lax.axis_index("i")
    x2 = x[0]
    k2 = K_ext[0]
    v2 = V_ext[0]
    wq_s = lax.dynamic_slice(Wq, (0, pos * D), (1024, D))
    wo_s = lax.dynamic_slice(Wo, (pos * D, 0), (D, 1024))

    out = pl.pallas_call(
        _body,
        out_shape=jax.ShapeDtypeStruct((SQ, 1024), jnp.float32),
        in_specs=[pl.BlockSpec(memory_space=pltpu.VMEM)] * 5,
        out_specs=pl.BlockSpec(memory_space=pltpu.VMEM),
        scratch_shapes=[
            pltpu.VMEM((N_DEV - 1, CHUNK, 1024), jnp.float32),
            pltpu.SemaphoreType.DMA((N_DEV - 1,)),
            pltpu.SemaphoreType.DMA((N_DEV - 1,)),
            pltpu.SemaphoreType.DMA((N_DEV - 1,)),
            pltpu.SemaphoreType.DMA((N_DEV - 1,)),
        ],
        compiler_params=pltpu.CompilerParams(collective_id=0),
    )(x2, wq_s, k2, v2, wo_s)
    return out[None]


# device time: 115883 ns/iter; 1.3173x vs baseline; 1.3173x over previous
import jax
import jax.numpy as jnp
from jax import lax
from jax.experimental import pallas as pl
from jax.experimental.pallas import tpu as pltpu

N_DEV = 8
SQ = 1024
D = 1024
HQ = 8
DH = 128
SCALE = 0.08838834764831843
CHUNK = SQ // N_DEV
R_HOPS = 4
L_HOPS = 3
F32 = jnp.float32


def _mod(v):
    return lax.rem(v + 2 * N_DEV, N_DEV)


def _body(x_ref, wq_ref, kT_ref, vT_ref, wo_ref, out_ref,
          rbuf_ref, qc_ref, rs_send, rs_recv, agr_send, agr_recv,
          agl_send, agl_recv):
    my_pos = lax.axis_index("i")
    left = _mod(my_pos - 1)
    right = _mod(my_pos + 1)

    barrier_sem = pltpu.get_barrier_semaphore()
    for nbr in (left, right):
        pl.semaphore_signal(
            barrier_sem, inc=1,
            device_id=(nbr,), device_id_type=pl.DeviceIdType.MESH,
        )
    pl.semaphore_wait(barrier_sem, 2)

    def compute_chunk(c):
        r0 = c * CHUNK
        xc = x_ref[pl.ds(r0, CHUNK), :]
        qc_ref[...] = jnp.dot(xc, wq_ref[...],
                              preferred_element_type=F32)
        rows = lax.broadcasted_iota(jnp.int32, (CHUNK, SQ), 0) + r0
        cols = lax.broadcasted_iota(jnp.int32, (CHUNK, SQ), 1)
        qb = rows // 64
        kb = cols // 64
        mask = (qb == kb) | (kb == 0) | (lax.rem(qb + kb, 3) == 0)

        def head_body(h, partial):
            q_h = qc_ref[:, pl.ds(h * DH, DH)]
            k_h = kT_ref[h]
            v_h = vT_ref[h]
            scores = lax.dot_general(
                q_h, k_h, (((1,), (1,)), ((), ())),
                preferred_element_type=F32) * SCALE
            scores = jnp.where(mask, scores, -1e9)
            m = jnp.max(scores, axis=1, keepdims=True)
            e = jnp.exp(scores - m)
            ssum = jnp.sum(e, axis=1, keepdims=True)
            w = e / ssum
            ctx_h = jnp.dot(w, v_h, preferred_element_type=F32)
            wo_h = wo_ref[pl.ds(h * DH, DH), :]
            return partial + jnp.dot(ctx_h, wo_h,
                                     preferred_element_type=F32)

        partial = lax.fori_loop(
            0, HQ, head_body, jnp.zeros((CHUNK, D), dtype=F32))
        out_ref[pl.ds(r0, CHUNK), :] = partial

    def rs_send_step(s, c):
        rdma = pltpu.make_async_remote_copy(
            src_ref=out_ref.at[pl.ds(c * CHUNK, CHUNK), :],
            dst_ref=rbuf_ref.at[s],
            send_sem=rs_send.at[s],
            recv_sem=rs_recv.at[s],
            device_id=(right,),
            device_id_type=pl.DeviceIdType.MESH,
        )
        rdma.start()

    def rs_recv_wait(s):
        rdma = pltpu.make_async_remote_copy(
            src_ref=out_ref.at[pl.ds(0, CHUNK), :],
            dst_ref=rbuf_ref.at[s],
            send_sem=rs_send.at[s],
            recv_sem=rs_recv.at[s],
            device_id=(right,),
            device_id_type=pl.DeviceIdType.MESH,
        )
        rdma.wait_recv()

    compute_chunk(my_pos)
    rs_send_step(0, my_pos)

    def rs_body(s, _):
        c = _mod(my_pos - s)
        compute_chunk(c)
        rs_recv_wait(s - 1)
        out_ref[pl.ds(c * CHUNK, CHUNK), :] = (
            out_ref[pl.ds(c * CHUNK, CHUNK), :] + rbuf_ref[s - 1]
        )
        rs_send_step(s, c)
        return 0

    lax.fori_loop(1, N_DEV - 1, rs_body, 0)

    c_own = _mod(my_pos + 1)
    compute_chunk(c_own)
    rs_recv_wait(N_DEV - 2)
    out_ref[pl.ds(c_own * CHUNK, CHUNK), :] = (
        out_ref[pl.ds(c_own * CHUNK, CHUNK), :] + rbuf_ref[N_DEV - 2]
    )

    def rs_drain(s, _):
        rdma = pltpu.make_async_remote_copy(
            src_ref=out_ref.at[pl.ds(0, CHUNK), :],
            dst_ref=rbuf_ref.at[s],
            send_sem=rs_send.at[s],
            recv_sem=rs_recv.at[s],
            device_id=(right,),
            device_id_type=pl.DeviceIdType.MESH,
        )
        rdma.wait_send()
        return 0

    lax.fori_loop(0, N_DEV - 1, rs_drain, 0)

    def ag_rdma(c, sems_send, sems_recv, s, dev):
        return pltpu.make_async_remote_copy(
            src_ref=out_ref.at[pl.ds(c * CHUNK, CHUNK), :],
            dst_ref=out_ref.at[pl.ds(c * CHUNK, CHUNK), :],
            send_sem=sems_send.at[s],
            recv_sem=sems_recv.at[s],
            device_id=(dev,),
            device_id_type=pl.DeviceIdType.MESH,
        )

    def ag_body(s, _):
        r_rdma = ag_rdma(_mod(my_pos + 1 - s), agr_send, agr_recv, s, right)
        r_rdma.start()
        l_rdma = ag_rdma(_mod(my_pos + 1 + s), agl_send, agl_recv, s, left)
        l_rdma.start()
        l_rdma.wait()
        r_rdma.wait()
        return 0

    lax.fori_loop(0, L_HOPS, ag_body, 0)

    s_last = R_HOPS - 1
    r_rdma = ag_rdma(_mod(my_pos + 1 - s_last), agr_send, agr_recv,
                     s_last, right)
    r_rdma.start()
    r_rdma.wait()


def kernel(x, Wq, K_ext, V_ext, Wo):
    pos = lax.axis_index("i")
    x2 = x[0]
    kT = jnp.transpose(K_ext[0], (1, 0, 2))
    vT = jnp.transpose(V_ext[0], (1, 0, 2))
    wq_s = lax.dynamic_slice(Wq, (0, pos * D), (1024, D))
    wo_s = lax.dynamic_slice(Wo, (pos * D, 0), (D, 1024))

    out = pl.pallas_call(
        _body,
        out_shape=jax.ShapeDtypeStruct((SQ, 1024), F32),
        in_specs=[pl.BlockSpec(memory_space=pltpu.VMEM)] * 5,
        out_specs=pl.BlockSpec(memory_space=pltpu.VMEM),
        scratch_shapes=[
            pltpu.VMEM((N_DEV - 1, CHUNK, 1024), F32),
            pltpu.VMEM((CHUNK, D), F32),
            pltpu.SemaphoreType.DMA((N_DEV - 1,)),
            pltpu.SemaphoreType.DMA((N_DEV - 1,)),
            pltpu.SemaphoreType.DMA((R_HOPS,)),
            pltpu.SemaphoreType.DMA((R_HOPS,)),
            pltpu.SemaphoreType.DMA((L_HOPS,)),
            pltpu.SemaphoreType.DMA((L_HOPS,)),
        ],
        compiler_params=pltpu.CompilerParams(collective_id=0),
    )(x2, wq_s, kT, vT, wo_s)
    return out[None]


# device time: 104109 ns/iter; 1.4663x vs baseline; 1.1131x over previous
import jax
import jax.numpy as jnp
from jax import lax
from jax.experimental import pallas as pl
from jax.experimental.pallas import tpu as pltpu

N_DEV = 8
SQ = 1024
D = 1024
HQ = 8
DH = 128
SCALE = 0.08838834764831843
CHUNK = SQ // N_DEV
R_HOPS = 4
L_HOPS = 3
F32 = jnp.float32
BF16 = jnp.bfloat16


def _mod(v):
    return lax.rem(v + 2 * N_DEV, N_DEV)


def _body(x_ref, wq_ref, kT_ref, vT_ref, wo_ref, out_ref,
          rbuf_ref, sbuf_ref, agr_buf, agl_buf, qc_ref,
          rs_send, rs_recv, agr_send, agr_recv, agl_send, agl_recv):
    my_pos = lax.axis_index("i")
    left = _mod(my_pos - 1)
    right = _mod(my_pos + 1)

    barrier_sem = pltpu.get_barrier_semaphore()
    for nbr in (left, right):
        pl.semaphore_signal(
            barrier_sem, inc=1,
            device_id=(nbr,), device_id_type=pl.DeviceIdType.MESH,
        )
    pl.semaphore_wait(barrier_sem, 2)

    def compute_chunk(c):
        r0 = c * CHUNK
        xc = x_ref[pl.ds(r0, CHUNK), :]
        qc_ref[...] = jnp.dot(xc, wq_ref[...],
                              preferred_element_type=F32)
        rows = lax.broadcasted_iota(jnp.int32, (CHUNK, SQ), 0) + r0
        cols = lax.broadcasted_iota(jnp.int32, (CHUNK, SQ), 1)
        qb = rows // 64
        kb = cols // 64
        mask = (qb == kb) | (kb == 0) | (lax.rem(qb + kb, 3) == 0)

        def head_body(h, partial):
            q_h = qc_ref[:, pl.ds(h * DH, DH)]
            k_h = kT_ref[h]
            v_h = vT_ref[h]
            scores = lax.dot_general(
                q_h, k_h, (((1,), (1,)), ((), ())),
                preferred_element_type=F32) * SCALE
            scores = jnp.where(mask, scores, -1e9)
            m = jnp.max(scores, axis=1, keepdims=True)
            e = jnp.exp(scores - m)
            ssum = jnp.sum(e, axis=1, keepdims=True)
            w = e / ssum
            ctx_h = jnp.dot(w, v_h, preferred_element_type=F32)
            wo_h = wo_ref[pl.ds(h * DH, DH), :]
            return partial + jnp.dot(ctx_h, wo_h,
                                     preferred_element_type=F32)

        partial = lax.fori_loop(
            0, HQ, head_body, jnp.zeros((CHUNK, D), dtype=F32))
        out_ref[pl.ds(r0, CHUNK), :] = partial

    def rs_send_step(s, c):
        sbuf_ref[s] = out_ref[pl.ds(c * CHUNK, CHUNK), :].astype(BF16)
        rdma = pltpu.make_async_remote_copy(
            src_ref=sbuf_ref.at[s],
            dst_ref=rbuf_ref.at[s],
            send_sem=rs_send.at[s],
            recv_sem=rs_recv.at[s],
            device_id=(right,),
            device_id_type=pl.DeviceIdType.MESH,
        )
        rdma.start()

    def rs_rdma(s):
        return pltpu.make_async_remote_copy(
            src_ref=sbuf_ref.at[s],
            dst_ref=rbuf_ref.at[s],
            send_sem=rs_send.at[s],
            recv_sem=rs_recv.at[s],
            device_id=(right,),
            device_id_type=pl.DeviceIdType.MESH,
        )

    compute_chunk(my_pos)
    rs_send_step(0, my_pos)

    def rs_body(s, _):
        c = _mod(my_pos - s)
        compute_chunk(c)
        rs_rdma(s - 1).wait_recv()
        out_ref[pl.ds(c * CHUNK, CHUNK), :] = (
            out_ref[pl.ds(c * CHUNK, CHUNK), :]
            + rbuf_ref[s - 1].astype(F32)
        )
        rs_send_step(s, c)
        return 0

    lax.fori_loop(1, N_DEV - 1, rs_body, 0)

    c_own = _mod(my_pos + 1)
    compute_chunk(c_own)
    rs_rdma(N_DEV - 2).wait_recv()
    out_ref[pl.ds(c_own * CHUNK, CHUNK), :] = (
        out_ref[pl.ds(c_own * CHUNK, CHUNK), :]
        + rbuf_ref[N_DEV - 2].astype(F32)
    )

    def rs_drain(s, _):
        rs_rdma(s).wait_send()
        return 0

    lax.fori_loop(0, N_DEV - 1, rs_drain, 0)

    sbuf_ref[0] = out_ref[pl.ds(c_own * CHUNK, CHUNK), :].astype(BF16)

    def ag_rdma(src, sems_send, sems_recv, s, dst, dev):
        return pltpu.make_async_remote_copy(
            src_ref=src,
            dst_ref=dst,
            send_sem=sems_send.at[s],
            recv_sem=sems_recv.at[s],
            device_id=(dev,),
            device_id_type=pl.DeviceIdType.MESH,
        )

    def ag_step(s, r_src, l_src):
        r_rdma = ag_rdma(r_src, agr_send, agr_recv, s,
                         agr_buf.at[s], right)
        r_rdma.start()
        if l_src is not None:
            l_rdma = ag_rdma(l_src, agl_send, agl_recv, s,
                             agl_buf.at[s], left)
            l_rdma.start()
            l_rdma.wait_recv()
            out_ref[pl.ds(_mod(my_pos + 2 + s) * CHUNK, CHUNK), :] = (
                agl_buf[s].astype(F32))
        r_rdma.wait_recv()
        out_ref[pl.ds(_mod(my_pos - s) * CHUNK, CHUNK), :] = (
            agr_buf[s].astype(F32))

    ag_step(0, sbuf_ref.at[0], sbuf_ref.at[0])

    def ag_body(s, _):
        ag_step(s, agr_buf.at[s - 1], agl_buf.at[s - 1])
        return 0

    lax.fori_loop(1, L_HOPS, ag_body, 0)

    ag_step(R_HOPS - 1, agr_buf.at[R_HOPS - 2], None)

    def agr_drain(s, _):
        ag_rdma(agr_buf.at[0], agr_send, agr_recv, s,
                agr_buf.at[s], right).wait_send()
        return 0

    def agl_drain(s, _):
        ag_rdma(agl_buf.at[0], agl_send, agl_recv, s,
                agl_buf.at[s], left).wait_send()
        return 0

    lax.fori_loop(0, R_HOPS, agr_drain, 0)
    lax.fori_loop(0, L_HOPS, agl_drain, 0)


def kernel(x, Wq, K_ext, V_ext, Wo):
    pos = lax.axis_index("i")
    x2 = x[0]
    kT = jnp.transpose(K_ext[0], (1, 0, 2))
    vT = jnp.transpose(V_ext[0], (1, 0, 2))
    wq_s = lax.dynamic_slice(Wq, (0, pos * D), (1024, D))
    wo_s = lax.dynamic_slice(Wo, (pos * D, 0), (D, 1024))

    out = pl.pallas_call(
        _body,
        out_shape=jax.ShapeDtypeStruct((SQ, 1024), F32),
        in_specs=[pl.BlockSpec(memory_space=pltpu.VMEM)] * 5,
        out_specs=pl.BlockSpec(memory_space=pltpu.VMEM),
        scratch_shapes=[
            pltpu.VMEM((N_DEV - 1, CHUNK, 1024), BF16),
            pltpu.VMEM((N_DEV - 1, CHUNK, 1024), BF16),
            pltpu.VMEM((R_HOPS, CHUNK, 1024), BF16),
            pltpu.VMEM((L_HOPS, CHUNK, 1024), BF16),
            pltpu.VMEM((CHUNK, D), F32),
            pltpu.SemaphoreType.DMA((N_DEV - 1,)),
            pltpu.SemaphoreType.DMA((N_DEV - 1,)),
            pltpu.SemaphoreType.DMA((R_HOPS,)),
            pltpu.SemaphoreType.DMA((R_HOPS,)),
            pltpu.SemaphoreType.DMA((L_HOPS,)),
            pltpu.SemaphoreType.DMA((L_HOPS,)),
        ],
        compiler_params=pltpu.CompilerParams(collective_id=0),
    )(x2, wq_s, kT, vT, wo_s)
    return out[None]


# device time: 100751 ns/iter; 1.5152x vs baseline; 1.0333x over previous
import jax
import jax.numpy as jnp
from jax import lax
from jax.experimental import pallas as pl
from jax.experimental.pallas import tpu as pltpu

N_DEV = 8
SQ = 1024
D = 1024
HQ = 8
DH = 128
SCALE = 0.08838834764831843
CHUNK = SQ // N_DEV
R_HOPS = 4
L_HOPS = 3
F32 = jnp.float32
BF16 = jnp.bfloat16


def _mod(v):
    return lax.rem(v + 2 * N_DEV, N_DEV)


def _body(x_ref, wq_ref, kT_ref, vT_ref, wo_ref, out_ref,
          rbuf_ref, sbuf_ref, agr_buf, agl_buf, qc_ref,
          rs_send, rs_recv, agr_send, agr_recv, agl_send, agl_recv):
    my_pos = lax.axis_index("i")
    left = _mod(my_pos - 1)
    right = _mod(my_pos + 1)

    barrier_sem = pltpu.get_barrier_semaphore()
    for nbr in (left, right):
        pl.semaphore_signal(
            barrier_sem, inc=1,
            device_id=(nbr,), device_id_type=pl.DeviceIdType.MESH,
        )
    pl.semaphore_wait(barrier_sem, 2)

    def compute_chunk(c):
        r0 = c * CHUNK
        xc = x_ref[pl.ds(r0, CHUNK), :]
        qc_ref[...] = jnp.dot(xc, wq_ref[...],
                              preferred_element_type=F32).astype(BF16)
        rows = lax.broadcasted_iota(jnp.int32, (CHUNK, SQ), 0) + r0
        cols = lax.broadcasted_iota(jnp.int32, (CHUNK, SQ), 1)
        qb = rows // 64
        kb = cols // 64
        mask = (qb == kb) | (kb == 0) | (lax.rem(qb + kb, 3) == 0)

        def head_body(h, partial):
            q_h = qc_ref[:, pl.ds(h * DH, DH)]
            k_h = kT_ref[h]
            v_h = vT_ref[h]
            scores = lax.dot_general(
                q_h, k_h, (((1,), (1,)), ((), ())),
                preferred_element_type=F32) * SCALE
            scores = jnp.where(mask, scores, -1e9)
            m = jnp.max(scores, axis=1, keepdims=True)
            e = jnp.exp(scores - m)
            ssum = jnp.sum(e, axis=1, keepdims=True)
            w = e / ssum
            ctx_h = jnp.dot(w, v_h, preferred_element_type=F32)
            wo_h = wo_ref[pl.ds(h * DH, DH), :]
            return partial + jnp.dot(ctx_h.astype(BF16), wo_h,
                                     preferred_element_type=F32)

        partial = lax.fori_loop(
            0, HQ, head_body, jnp.zeros((CHUNK, D), dtype=F32))
        out_ref[pl.ds(r0, CHUNK), :] = partial

    def rs_send_step(s, c):
        sbuf_ref[s] = out_ref[pl.ds(c * CHUNK, CHUNK), :].astype(BF16)
        rdma = pltpu.make_async_remote_copy(
            src_ref=sbuf_ref.at[s],
            dst_ref=rbuf_ref.at[s],
            send_sem=rs_send.at[s],
            recv_sem=rs_recv.at[s],
            device_id=(right,),
            device_id_type=pl.DeviceIdType.MESH,
        )
        rdma.start()

    def rs_rdma(s):
        return pltpu.make_async_remote_copy(
            src_ref=sbuf_ref.at[s],
            dst_ref=rbuf_ref.at[s],
            send_sem=rs_send.at[s],
            recv_sem=rs_recv.at[s],
            device_id=(right,),
            device_id_type=pl.DeviceIdType.MESH,
        )

    compute_chunk(my_pos)
    rs_send_step(0, my_pos)

    def rs_body(s, _):
        c = _mod(my_pos - s)
        compute_chunk(c)
        rs_rdma(s - 1).wait_recv()
        out_ref[pl.ds(c * CHUNK, CHUNK), :] = (
            out_ref[pl.ds(c * CHUNK, CHUNK), :]
            + rbuf_ref[s - 1].astype(F32)
        )
        rs_send_step(s, c)
        return 0

    lax.fori_loop(1, N_DEV - 1, rs_body, 0)

    c_own = _mod(my_pos + 1)
    compute_chunk(c_own)
    rs_rdma(N_DEV - 2).wait_recv()
    out_ref[pl.ds(c_own * CHUNK, CHUNK), :] = (
        out_ref[pl.ds(c_own * CHUNK, CHUNK), :]
        + rbuf_ref[N_DEV - 2].astype(F32)
    )

    def rs_drain(s, _):
        rs_rdma(s).wait_send()
        return 0

    lax.fori_loop(0, N_DEV - 1, rs_drain, 0)

    sbuf_ref[0] = out_ref[pl.ds(c_own * CHUNK, CHUNK), :].astype(BF16)

    def ag_rdma(src, sems_send, sems_recv, s, dst, dev):
        return pltpu.make_async_remote_copy(
            src_ref=src,
            dst_ref=dst,
            send_sem=sems_send.at[s],
            recv_sem=sems_recv.at[s],
            device_id=(dev,),
            device_id_type=pl.DeviceIdType.MESH,
        )

    def ag_start(src, sems_send, sems_recv, s, dst, dev):
        rdma = ag_rdma(src, sems_send, sems_recv, s, dst, dev)
        rdma.start()

    def ag_wait_recv(sems_send, sems_recv, s, dst, dev):
        ag_rdma(sbuf_ref.at[0], sems_send, sems_recv, s, dst, dev).wait_recv()

    ag_start(sbuf_ref.at[0], agr_send, agr_recv, 0, agr_buf.at[0], right)
    ag_start(sbuf_ref.at[0], agl_send, agl_recv, 0, agl_buf.at[0], left)

    def ag_body(s, _):
        ag_wait_recv(agr_send, agr_recv, s - 1, agr_buf.at[s - 1], right)
        ag_start(agr_buf.at[s - 1], agr_send, agr_recv, s,
                 agr_buf.at[s], right)
        ag_wait_recv(agl_send, agl_recv, s - 1, agl_buf.at[s - 1], left)
        ag_start(agl_buf.at[s - 1], agl_send, agl_recv, s,
                 agl_buf.at[s], left)
        out_ref[pl.ds(_mod(my_pos - (s - 1)) * CHUNK, CHUNK), :] = (
            agr_buf[s - 1].astype(F32))
        out_ref[pl.ds(_mod(my_pos + 1 + s) * CHUNK, CHUNK), :] = (
            agl_buf[s - 1].astype(F32))
        return 0

    lax.fori_loop(1, L_HOPS, ag_body, 0)

    s3 = R_HOPS - 1
    ag_wait_recv(agr_send, agr_recv, s3 - 1, agr_buf.at[s3 - 1], right)
    ag_start(agr_buf.at[s3 - 1], agr_send, agr_recv, s3,
             agr_buf.at[s3], right)
    out_ref[pl.ds(_mod(my_pos - (s3 - 1)) * CHUNK, CHUNK), :] = (
        agr_buf[s3 - 1].astype(F32))
    ag_wait_recv(agl_send, agl_recv, L_HOPS - 1,
                 agl_buf.at[L_HOPS - 1], left)
    out_ref[pl.ds(_mod(my_pos + 1 + L_HOPS) * CHUNK, CHUNK), :] = (
        agl_buf[L_HOPS - 1].astype(F32))
    ag_wait_recv(agr_send, agr_recv, s3, agr_buf.at[s3], right)
    out_ref[pl.ds(_mod(my_pos - s3) * CHUNK, CHUNK), :] = (
        agr_buf[s3].astype(F32))

    def agr_drain(s, _):
        ag_rdma(agr_buf.at[0], agr_send, agr_recv, s,
                agr_buf.at[s], right).wait_send()
        return 0

    def agl_drain(s, _):
        ag_rdma(agl_buf.at[0], agl_send, agl_recv, s,
                agl_buf.at[s], left).wait_send()
        return 0

    lax.fori_loop(0, R_HOPS, agr_drain, 0)
    lax.fori_loop(0, L_HOPS, agl_drain, 0)


def kernel(x, Wq, K_ext, V_ext, Wo):
    pos = lax.axis_index("i")
    x2 = x[0].astype(BF16)
    kT = jnp.transpose(K_ext[0], (1, 0, 2)).astype(BF16)
    vT = jnp.transpose(V_ext[0], (1, 0, 2))
    wq_s = lax.dynamic_slice(Wq, (0, pos * D), (1024, D)).astype(BF16)
    wo_s = lax.dynamic_slice(Wo, (pos * D, 0), (D, 1024)).astype(BF16)

    out = pl.pallas_call(
        _body,
        out_shape=jax.ShapeDtypeStruct((SQ, 1024), F32),
        in_specs=[pl.BlockSpec(memory_space=pltpu.VMEM)] * 5,
        out_specs=pl.BlockSpec(memory_space=pltpu.VMEM),
        scratch_shapes=[
            pltpu.VMEM((N_DEV - 1, CHUNK, 1024), BF16),
            pltpu.VMEM((N_DEV - 1, CHUNK, 1024), BF16),
            pltpu.VMEM((R_HOPS, CHUNK, 1024), BF16),
            pltpu.VMEM((L_HOPS, CHUNK, 1024), BF16),
            pltpu.VMEM((CHUNK, D), BF16),
            pltpu.SemaphoreType.DMA((N_DEV - 1,)),
            pltpu.SemaphoreType.DMA((N_DEV - 1,)),
            pltpu.SemaphoreType.DMA((R_HOPS,)),
            pltpu.SemaphoreType.DMA((R_HOPS,)),
            pltpu.SemaphoreType.DMA((L_HOPS,)),
            pltpu.SemaphoreType.DMA((L_HOPS,)),
        ],
        compiler_params=pltpu.CompilerParams(collective_id=0),
    )(x2, wq_s, kT, vT, wo_s)
    return out[None]
